# baseline (device time: 35336 ns/iter reference)
import jax
import jax.numpy as jnp
from jax import lax
from jax.experimental import pallas as pl
from jax.experimental.pallas import tpu as pltpu

N_DEV = 4
B, Sq, D = 2, 256, 768
Hq, Dh = 8, 64
Dq = Hq * Dh
R = B * Sq
GROUP_ROWS = Dq + 2 * Hq
SCALE = 0.125
COMM_DTYPE = jnp.bfloat16


def kernel(x, Wq, Wo, K_ext, V_ext):
    Skv = K_ext.shape[1]

    def body(x_ref, wq_ref, wo_ref, k_ref, v_ref, out_ref,
             khv, vhv, pack_ref, recv_ref, send_sems, recv_sems, kv_sems):
        my = lax.axis_index("i")
        left = lax.rem(my + N_DEV - 1, N_DEV)
        right = lax.rem(my + 1, N_DEV)
        diag = lax.rem(my + 2, N_DEV)
        peers = [(left, 1), (right, 0), (diag, 2)]

        barrier = pltpu.get_barrier_semaphore()
        for peer, _ in peers:
            pl.semaphore_signal(barrier, inc=1, device_id=(peer,),
                                device_id_type=pl.DeviceIdType.MESH)

        cp_k = [[None] * Hq for _ in range(B)]
        cp_v = [[None] * Hq for _ in range(B)]
        for b in range(B):
            for h in range(Hq):
                cp_k[b][h] = pltpu.make_async_copy(
                    k_ref.at[b, :, h, :], khv.at[b, h], kv_sems.at[0, b, h])
                cp_v[b][h] = pltpu.make_async_copy(
                    v_ref.at[b, :, h, :], vhv.at[b, h], kv_sems.at[1, b, h])
                cp_k[b][h].start()
                cp_v[b][h].start()

        Q = jnp.dot(x_ref[:].astype(COMM_DTYPE), wq_ref[:].astype(COMM_DTYPE),
                    preferred_element_type=jnp.float32).astype(COMM_DTYPE)

        o_acc = [[None] * Hq for _ in range(B)]
        m_acc = [[None] * Hq for _ in range(B)]
        l_acc = [[None] * Hq for _ in range(B)]

        def compute_group(b):
            for h in range(Hq):
                q_bh = Q[b * Sq:(b + 1) * Sq, h * Dh:(h + 1) * Dh]
                cp_k[b][h].wait()
                cp_v[b][h].wait()
                k_bh = khv[b, h].astype(COMM_DTYPE)
                v_bh = vhv[b, h].astype(COMM_DTYPE)
                sT = lax.dot_general(
                    k_bh, q_bh, (((1,), (1,)), ((), ())),
                    preferred_element_type=jnp.float32) * SCALE
                m_row = jnp.max(sT, axis=0, keepdims=True)
                m_row = m_row.astype(COMM_DTYPE).astype(jnp.float32)
                pT = jnp.exp(sT - m_row)
                l_row = jnp.sum(pT, axis=0, keepdims=True)
                oT = lax.dot_general(
                    v_bh, pT.astype(COMM_DTYPE), (((0,), (0,)), ((), ())),
                    preferred_element_type=jnp.float32)
                o_acc[b][h], m_acc[b][h], l_acc[b][h] = oT, m_row, l_row
                pack_ref[b, h * Dh:(h + 1) * Dh, :] = oT.astype(COMM_DTYPE)
                pack_ref[b, Dq + h:Dq + h + 1, :] = m_row.astype(COMM_DTYPE)
                pack_ref[b, Dq + Hq + h:Dq + Hq + h + 1, :] = (
                    l_row.astype(COMM_DTYPE))

        def send_piece(b):
            ds = []
            for i, (peer, slot) in enumerate(peers):
                d_ = pltpu.make_async_remote_copy(
                    src_ref=pack_ref.at[b],
                    dst_ref=recv_ref.at[slot, b],
                    send_sem=send_sems.at[i, b],
                    recv_sem=recv_sems.at[i, b],
                    device_id=(peer,),
                    device_id_type=pl.DeviceIdType.MESH,
                )
                d_.start()
                ds.append(d_)
            return ds

        def merge_piece(b, slot):
            recv = recv_ref[slot, b].astype(jnp.float32)
            for h in range(Hq):
                o_r = recv[h * Dh:(h + 1) * Dh, :]
                m_r = recv[Dq + h:Dq + h + 1, :]
                l_r = recv[Dq + Hq + h:Dq + Hq + h + 1, :]
                m_new = jnp.maximum(m_acc[b][h], m_r)
                a_old = jnp.exp(m_acc[b][h] - m_new)
                a_new = jnp.exp(m_r - m_new)
                l_acc[b][h] = l_acc[b][h] * a_old + l_r * a_new
                o_acc[b][h] = o_acc[b][h] * a_old + o_r * a_new
                m_acc[b][h] = m_new

        compute_group(0)
        pl.semaphore_wait(barrier, 3)
        descs = send_piece(0)
        compute_group(1)
        descs += send_piece(1)

        for b in range(B):
            for i, (_, slot) in enumerate(peers):
                descs[3 * b + i].wait_recv()
                merge_piece(b, slot)

        attT = jnp.concatenate(
            [jnp.concatenate(
                [o_acc[b][h] / l_acc[b][h] for h in range(Hq)], axis=0)
             for b in range(B)],
            axis=1,
        )
        out_ref[:] = lax.dot_general(
            attT.astype(COMM_DTYPE), wo_ref[:].astype(COMM_DTYPE),
            (((0,), (0,)), ((), ())),
            preferred_element_type=jnp.float32)

        for d_ in descs:
            d_.wait_send()

    out2d = pl.pallas_call(
        body,
        out_shape=jax.ShapeDtypeStruct((R, D), jnp.float32),
        in_specs=[pl.BlockSpec(memory_space=pltpu.VMEM)] * 5,
        out_specs=pl.BlockSpec(memory_space=pltpu.VMEM),
        scratch_shapes=[
            pltpu.VMEM((B, Hq, 512, Dh), jnp.float32),
            pltpu.VMEM((B, Hq, 512, Dh), jnp.float32),
            pltpu.VMEM((B, GROUP_ROWS, Sq), COMM_DTYPE),
            pltpu.VMEM((3, B, GROUP_ROWS, Sq), COMM_DTYPE),
            pltpu.SemaphoreType.DMA((3, B)),
            pltpu.SemaphoreType.DMA((3, B)),
            pltpu.SemaphoreType.DMA((2, B, Hq)),
        ],
        compiler_params=pltpu.CompilerParams(collective_id=0),
    )(x.reshape(R, D), Wq, Wo, K_ext, V_ext)

    return out2d.reshape(B, Sq, D)


# device time: 35307 ns/iter; 1.0008x vs baseline; 1.0008x over previous
import jax
import jax.numpy as jnp
from jax import lax
from jax.experimental import pallas as pl
from jax.experimental.pallas import tpu as pltpu

N_DEV = 4
B, Sq, D = 2, 256, 768
Hq, Dh = 8, 64
Dq = Hq * Dh
R = B * Sq
GROUP_ROWS = Dq + 2 * Hq
SCALE = 0.125
COMM_DTYPE = jnp.bfloat16


def kernel(x, Wq, Wo, K_ext, V_ext):
    Skv = K_ext.shape[1]

    def body(x_ref, wq_ref, wo_ref, k_ref, v_ref, out_ref,
             khv, vhv, pack_ref, recv_ref, send_sems, recv_sems, kv_sems):
        my = lax.axis_index("i")
        left = lax.rem(my + N_DEV - 1, N_DEV)
        right = lax.rem(my + 1, N_DEV)
        diag = lax.rem(my + 2, N_DEV)
        peers = [(left, 1), (right, 0), (diag, 2)]

        barrier = pltpu.get_barrier_semaphore()
        for peer, _ in peers:
            pl.semaphore_signal(barrier, inc=1, device_id=(peer,),
                                device_id_type=pl.DeviceIdType.MESH)

        cp_k = [[None] * Hq for _ in range(B)]
        cp_v = [[None] * Hq for _ in range(B)]
        for b in range(B):
            for h in range(Hq):
                cp_k[b][h] = pltpu.make_async_copy(
                    k_ref.at[b, :, h, :], khv.at[b, h], kv_sems.at[0, b, h])
                cp_v[b][h] = pltpu.make_async_copy(
                    v_ref.at[b, :, h, :], vhv.at[b, h], kv_sems.at[1, b, h])
                cp_k[b][h].start()
                cp_v[b][h].start()

        Q = jnp.dot(x_ref[:].astype(COMM_DTYPE), wq_ref[:].astype(COMM_DTYPE),
                    preferred_element_type=jnp.float32).astype(COMM_DTYPE)

        o_acc = [[None] * Hq for _ in range(B)]
        m_acc = [[None] * Hq for _ in range(B)]
        l_acc = [[None] * Hq for _ in range(B)]

        def compute_group(b):
            for h in range(Hq):
                q_bh = Q[b * Sq:(b + 1) * Sq, h * Dh:(h + 1) * Dh]
                cp_k[b][h].wait()
                cp_v[b][h].wait()
                k_bh = khv[b, h].astype(COMM_DTYPE)
                v_bh = vhv[b, h].astype(COMM_DTYPE)
                sT = lax.dot_general(
                    k_bh, q_bh, (((1,), (1,)), ((), ())),
                    preferred_element_type=jnp.float32) * SCALE
                m_row = jnp.max(sT, axis=0, keepdims=True)
                m_row = m_row.astype(COMM_DTYPE).astype(jnp.float32)
                pT = jnp.exp(sT - m_row)
                l_row = jnp.sum(pT, axis=0, keepdims=True)
                oT = lax.dot_general(
                    v_bh, pT.astype(COMM_DTYPE), (((0,), (0,)), ((), ())),
                    preferred_element_type=jnp.float32)
                o_acc[b][h], m_acc[b][h], l_acc[b][h] = oT, m_row, l_row
                pack_ref[b, h * Dh:(h + 1) * Dh, :] = oT.astype(COMM_DTYPE)
                pack_ref[b, Dq + h:Dq + h + 1, :] = m_row.astype(COMM_DTYPE)
                pack_ref[b, Dq + Hq + h:Dq + Hq + h + 1, :] = (
                    l_row.astype(COMM_DTYPE))

        def send_piece(b):
            ds = []
            for i, (peer, slot) in enumerate(peers):
                d_ = pltpu.make_async_remote_copy(
                    src_ref=pack_ref.at[b],
                    dst_ref=recv_ref.at[slot, b],
                    send_sem=send_sems.at[i, b],
                    recv_sem=recv_sems.at[i, b],
                    device_id=(peer,),
                    device_id_type=pl.DeviceIdType.MESH,
                )
                d_.start()
                ds.append(d_)
            return ds

        def merge_piece(b, slot):
            recv = recv_ref[slot, b].astype(jnp.float32)
            for h in range(Hq):
                o_r = recv[h * Dh:(h + 1) * Dh, :]
                m_r = recv[Dq + h:Dq + h + 1, :]
                l_r = recv[Dq + Hq + h:Dq + Hq + h + 1, :]
                m_new = jnp.maximum(m_acc[b][h], m_r)
                a_old = jnp.exp(m_acc[b][h] - m_new)
                a_new = jnp.exp(m_r - m_new)
                l_acc[b][h] = l_acc[b][h] * a_old + l_r * a_new
                o_acc[b][h] = o_acc[b][h] * a_old + o_r * a_new
                m_acc[b][h] = m_new

        compute_group(0)
        pl.semaphore_wait(barrier, 3)
        descs = send_piece(0)
        compute_group(1)
        descs += send_piece(1)

        for b in range(B):
            for i, (_, slot) in enumerate(peers):
                descs[3 * b + i].wait_recv()
                merge_piece(b, slot)

        attT = jnp.concatenate(
            [jnp.concatenate(
                [o_acc[b][h] / l_acc[b][h] for h in range(Hq)], axis=0)
             for b in range(B)],
            axis=1,
        )
        out_ref[:] = lax.dot_general(
            attT.astype(COMM_DTYPE), wo_ref[:].astype(COMM_DTYPE),
            (((0,), (0,)), ((), ())),
            preferred_element_type=jnp.float32)

        for d_ in descs:
            d_.wait_send()

    out2d = pl.pallas_call(
        body,
        out_shape=jax.ShapeDtypeStruct((R, D), jnp.float32),
        in_specs=[pl.BlockSpec(memory_space=pltpu.VMEM)] * 3
        + [pl.BlockSpec(memory_space=pltpu.MemorySpace.HBM)] * 2,
        out_specs=pl.BlockSpec(memory_space=pltpu.VMEM),
        scratch_shapes=[
            pltpu.VMEM((B, Hq, 512, Dh), jnp.float32),
            pltpu.VMEM((B, Hq, 512, Dh), jnp.float32),
            pltpu.VMEM((B, GROUP_ROWS, Sq), COMM_DTYPE),
            pltpu.VMEM((3, B, GROUP_ROWS, Sq), COMM_DTYPE),
            pltpu.SemaphoreType.DMA((3, B)),
            pltpu.SemaphoreType.DMA((3, B)),
            pltpu.SemaphoreType.DMA((2, B, Hq)),
        ],
        compiler_params=pltpu.CompilerParams(collective_id=0),
    )(x.reshape(R, D), Wq, Wo, K_ext, V_ext)

    return out2d.reshape(B, Sq, D)


# device time: 30710 ns/iter; 1.1506x vs baseline; 1.1497x over previous
import jax
import jax.numpy as jnp
from jax import lax
from jax.experimental import pallas as pl
from jax.experimental.pallas import tpu as pltpu

N_DEV = 4
B, Sq, D = 2, 256, 768
Hq, Dh = 8, 64
Dq = Hq * Dh
R = B * Sq
H_PER = 4
N_HALF = Hq // H_PER
HALF_ROWS = H_PER * Dh + 16
M_ROW = H_PER * Dh
L_ROW = M_ROW + 8
SCALE = 0.125
COMM_DTYPE = jnp.bfloat16


def kernel(x, Wq, Wo, K_ext, V_ext):
    Skv = K_ext.shape[1]

    xf = x.reshape(R, D)
    Kf = K_ext.reshape(B * Skv, Dq)
    Vf = V_ext.reshape(B * Skv, Dq)

    def body(x_hbm, wq_hbm, wo_hbm, k_hbm, v_hbm, out_ref,
             xv, wqv, wov, kv, vv,
             pack_ref, recv_ref, send_sems, recv_sems, in_sems):
        my = lax.axis_index("i")
        left = lax.rem(my + N_DEV - 1, N_DEV)
        right = lax.rem(my + 1, N_DEV)
        diag = lax.rem(my + 2, N_DEV)
        peers = [(left, 1), (right, 0), (diag, 2)]

        barrier = pltpu.get_barrier_semaphore()
        for peer, _ in peers:
            pl.semaphore_signal(barrier, inc=1, device_id=(peer,),
                                device_id_type=pl.DeviceIdType.MESH)

        cp_x = pltpu.make_async_copy(x_hbm, xv, in_sems.at[0])
        cp_wq = pltpu.make_async_copy(wq_hbm, wqv, in_sems.at[1])
        cp_k0 = pltpu.make_async_copy(k_hbm.at[pl.ds(0, Skv)],
                                      kv.at[pl.ds(0, Skv)], in_sems.at[2])
        cp_v0 = pltpu.make_async_copy(v_hbm.at[pl.ds(0, Skv)],
                                      vv.at[pl.ds(0, Skv)], in_sems.at[3])
        cp_k1 = pltpu.make_async_copy(k_hbm.at[pl.ds(Skv, Skv)],
                                      kv.at[pl.ds(Skv, Skv)], in_sems.at[4])
        cp_v1 = pltpu.make_async_copy(v_hbm.at[pl.ds(Skv, Skv)],
                                      vv.at[pl.ds(Skv, Skv)], in_sems.at[5])
        cp_wo = pltpu.make_async_copy(wo_hbm, wov, in_sems.at[6])
        for cp in (cp_x, cp_wq, cp_k0, cp_v0, cp_k1, cp_v1, cp_wo):
            cp.start()

        cp_x.wait()
        cp_wq.wait()
        Q = jnp.dot(xv[:].astype(COMM_DTYPE), wqv[:].astype(COMM_DTYPE),
                    preferred_element_type=jnp.float32).astype(COMM_DTYPE)

        o_acc = [[None] * Hq for _ in range(B)]
        m_acc = [[None] * Hq for _ in range(B)]
        l_acc = [[None] * Hq for _ in range(B)]

        def compute_half(b, hh):
            Kb = kv[b * Skv:(b + 1) * Skv, :].astype(COMM_DTYPE)
            Vb = vv[b * Skv:(b + 1) * Skv, :].astype(COMM_DTYPE)
            for j in range(H_PER):
                h = hh * H_PER + j
                q_bh = Q[b * Sq:(b + 1) * Sq, h * Dh:(h + 1) * Dh]
                k_bh = Kb[:, h * Dh:(h + 1) * Dh]
                v_bh = Vb[:, h * Dh:(h + 1) * Dh]
                sT = lax.dot_general(
                    k_bh, q_bh, (((1,), (1,)), ((), ())),
                    preferred_element_type=jnp.float32) * SCALE
                m_row = jnp.max(sT, axis=0, keepdims=True)
                m_row = m_row.astype(COMM_DTYPE).astype(jnp.float32)
                pT = jnp.exp(sT - m_row)
                l_row = jnp.sum(pT, axis=0, keepdims=True)
                oT = lax.dot_general(
                    v_bh, pT.astype(COMM_DTYPE), (((0,), (0,)), ((), ())),
                    preferred_element_type=jnp.float32)
                o_acc[b][h], m_acc[b][h], l_acc[b][h] = oT, m_row, l_row
                pack_ref[b, hh, j * Dh:(j + 1) * Dh, :] = oT.astype(COMM_DTYPE)
                pack_ref[b, hh, M_ROW + j:M_ROW + j + 1, :] = (
                    m_row.astype(COMM_DTYPE))
                pack_ref[b, hh, L_ROW + j:L_ROW + j + 1, :] = (
                    l_row.astype(COMM_DTYPE))

        def send_half(b, hh):
            ds = []
            for i, (peer, slot) in enumerate(peers):
                d_ = pltpu.make_async_remote_copy(
                    src_ref=pack_ref.at[b, hh],
                    dst_ref=recv_ref.at[slot, b, hh],
                    send_sem=send_sems.at[i, b, hh],
                    recv_sem=recv_sems.at[i, b, hh],
                    device_id=(peer,),
                    device_id_type=pl.DeviceIdType.MESH,
                )
                d_.start()
                ds.append(d_)
            return ds

        def merge_half(b, hh, slot):
            recv = recv_ref[slot, b, hh].astype(jnp.float32)
            for j in range(H_PER):
                h = hh * H_PER + j
                o_r = recv[j * Dh:(j + 1) * Dh, :]
                m_r = recv[M_ROW + j:M_ROW + j + 1, :]
                l_r = recv[L_ROW + j:L_ROW + j + 1, :]
                m_new = jnp.maximum(m_acc[b][h], m_r)
                a_old = jnp.exp(m_acc[b][h] - m_new)
                a_new = jnp.exp(m_r - m_new)
                l_acc[b][h] = l_acc[b][h] * a_old + l_r * a_new
                o_acc[b][h] = o_acc[b][h] * a_old + o_r * a_new
                m_acc[b][h] = m_new

        cp_k0.wait()
        cp_v0.wait()
        compute_half(0, 0)
        pl.semaphore_wait(barrier, 3)
        descs = {(0, 0): send_half(0, 0)}
        compute_half(0, 1)
        descs[(0, 1)] = send_half(0, 1)
        cp_k1.wait()
        cp_v1.wait()
        compute_half(1, 0)
        descs[(1, 0)] = send_half(1, 0)
        compute_half(1, 1)
        descs[(1, 1)] = send_half(1, 1)

        for b in range(B):
            for hh in range(N_HALF):
                for i, (_, slot) in enumerate(peers):
                    descs[(b, hh)][i].wait_recv()
                    merge_half(b, hh, slot)

        attT = jnp.concatenate(
            [jnp.concatenate(
                [o_acc[b][h] / l_acc[b][h] for h in range(Hq)], axis=0)
             for b in range(B)],
            axis=1,
        )
        cp_wo.wait()
        out_ref[:] = lax.dot_general(
            attT.astype(COMM_DTYPE), wov[:].astype(COMM_DTYPE),
            (((0,), (0,)), ((), ())),
            preferred_element_type=jnp.float32)

        for ds in descs.values():
            for d_ in ds:
                d_.wait_send()

    out2d = pl.pallas_call(
        body,
        out_shape=jax.ShapeDtypeStruct((R, D), jnp.float32),
        in_specs=[pl.BlockSpec(memory_space=pltpu.MemorySpace.HBM)] * 5,
        out_specs=pl.BlockSpec(memory_space=pltpu.VMEM),
        scratch_shapes=[
            pltpu.VMEM((R, D), jnp.float32),
            pltpu.VMEM((D, Dq), jnp.float32),
            pltpu.VMEM((Dq, D), jnp.float32),
            pltpu.VMEM((B * 512, Dq), jnp.float32),
            pltpu.VMEM((B * 512, Dq), jnp.float32),
            pltpu.VMEM((B, N_HALF, HALF_ROWS, Sq), COMM_DTYPE),
            pltpu.VMEM((3, B, N_HALF, HALF_ROWS, Sq), COMM_DTYPE),
            pltpu.SemaphoreType.DMA((3, B, N_HALF)),
            pltpu.SemaphoreType.DMA((3, B, N_HALF)),
            pltpu.SemaphoreType.DMA((7,)),
        ],
        compiler_params=pltpu.CompilerParams(collective_id=0),
    )(xf, Wq, Wo, Kf, Vf)

    return out2d.reshape(B, Sq, D)
